# Initial kernel scaffold; baseline (speedup 1.0000x reference)
import functools

import jax
import jax.numpy as jnp
from jax import lax
from jax.experimental import pallas as pl
from jax.experimental.pallas import tpu as pltpu
from jax.experimental.pallas import tpu_sc as plsc

N = 10000
E = 320000
D = 128
HALF = 64
OUT = 128

NC = 2
NT = 16
LANES = 16

GROUP = 128
GROUPS_PER_CHUNK = 8
CHUNK = GROUP * GROUPS_PER_CHUNK
CHUNKS_PER_TILE = 20
EDGES_PER_TILE = CHUNK * CHUNKS_PER_TILE
E_PAD = EDGES_PER_TILE * NT
ROWS_PER_TILE = N // NT


def _spmm2_sc(xs, col2, row2, adjf):
    mesh = plsc.VectorSubcoreMesh(core_axis_name="c", subcore_axis_name="s")
    y_t = jax.ShapeDtypeStruct((NC * N, HALF), jnp.float32)

    @functools.partial(
        pl.kernel,
        out_type=[y_t, y_t],
        mesh=mesh,
        scratch_types=[
            pltpu.VMEM((GROUPS_PER_CHUNK, GROUP), jnp.int32),
            pltpu.VMEM((GROUPS_PER_CHUNK, GROUP), jnp.int32),
            pltpu.VMEM((CHUNK,), jnp.float32),
            pltpu.VMEM((CHUNK, HALF), jnp.float32),
            pltpu.VMEM_SHARED((N, HALF), jnp.float32),
            pltpu.VMEM_SHARED((N, HALF), jnp.float32),
            pltpu.SemaphoreType.DMA,
        ],
    )
    def k(xs_hbm, col_hbm, row_hbm, adj_hbm, y1_hbm, y2_hbm,
          colv, rowv, adjv, buf, acc1, acc2, sem):
        core = lax.axis_index("c")
        tid = lax.axis_index("s")
        src_off = core * N

        @pl.loop(0, ROWS_PER_TILE)
        def _(r):
            for c in range(HALF // LANES):
                buf[r, pl.ds(c * LANES, LANES)] = jnp.zeros((LANES,), jnp.float32)

        pltpu.sync_copy(buf.at[pl.ds(0, ROWS_PER_TILE)],
                        acc1.at[pl.ds(tid * ROWS_PER_TILE, ROWS_PER_TILE)])
        pltpu.sync_copy(buf.at[pl.ds(0, ROWS_PER_TILE)],
                        acc2.at[pl.ds(tid * ROWS_PER_TILE, ROWS_PER_TILE)])
        plsc.subcore_barrier()

        def hop(src_hbm, out_hbm, acc):
            @pl.loop(0, CHUNKS_PER_TILE)
            def _(s):
                base_g = (tid * CHUNKS_PER_TILE + s) * GROUPS_PER_CHUNK
                pltpu.sync_copy(col_hbm.at[pl.ds(base_g, GROUPS_PER_CHUNK)], colv)
                pltpu.sync_copy(row_hbm.at[pl.ds(base_g, GROUPS_PER_CHUNK)], rowv)
                pltpu.sync_copy(adj_hbm.at[pl.ds(base_g * GROUP, CHUNK)], adjv)

                @pl.loop(0, GROUPS_PER_CHUNK)
                def _(j):
                    for c in range(GROUP // LANES):
                        sl = pl.ds(c * LANES, LANES)
                        colv[j, sl] = colv[j, sl] + src_off

                waits = []
                for j in range(GROUPS_PER_CHUNK):
                    waits.append(pltpu.async_copy(
                        src_hbm.at[colv.at[j]],
                        buf.at[pl.ds(j * GROUP, GROUP)], sem))
                for w in waits:
                    w.wait()

                @pl.loop(0, CHUNK // LANES)
                def _(g):
                    for l in range(LANES):
                        e = g * LANES + l
                        a = adjv[e]
                        for c in range(HALF // LANES):
                            sl = pl.ds(c * LANES, LANES)
                            buf[e, sl] = buf[e, sl] * a

                for j in range(GROUPS_PER_CHUNK):
                    pltpu.sync_copy(buf.at[pl.ds(j * GROUP, GROUP)],
                                    acc.at[rowv.at[j]], add=True)

            plsc.subcore_barrier()
            pltpu.sync_copy(
                acc.at[pl.ds(tid * ROWS_PER_TILE, ROWS_PER_TILE)],
                out_hbm.at[pl.ds(src_off + tid * ROWS_PER_TILE, ROWS_PER_TILE)])
            plsc.subcore_barrier()

        hop(xs_hbm, y1_hbm, acc1)
        hop(y1_hbm, y2_hbm, acc2)

    return k(xs, col2, row2, adjf)


def _linear_tc(x, y1f, y2f, W, b2):
    BN = 1000

    def body(x_ref, y1a, y1b, y2a, y2b, w_ref, b_ref, o_ref):
        p = jax.lax.Precision.HIGHEST
        acc = jnp.dot(x_ref[...], w_ref[0:D, :], precision=p,
                      preferred_element_type=jnp.float32)
        acc += jnp.dot(y1a[...], w_ref[D:D + HALF, :], precision=p,
                       preferred_element_type=jnp.float32)
        acc += jnp.dot(y1b[...], w_ref[D + HALF:2 * D, :], precision=p,
                       preferred_element_type=jnp.float32)
        acc += jnp.dot(y2a[...], w_ref[2 * D:2 * D + HALF, :], precision=p,
                       preferred_element_type=jnp.float32)
        acc += jnp.dot(y2b[...], w_ref[2 * D + HALF:3 * D, :], precision=p,
                       preferred_element_type=jnp.float32)
        o_ref[...] = acc + b_ref[...]

    nb = N // BN
    return pl.pallas_call(
        body,
        grid=(nb,),
        in_specs=[
            pl.BlockSpec((BN, D), lambda i: (i, 0)),
            pl.BlockSpec((BN, HALF), lambda i: (i, 0)),
            pl.BlockSpec((BN, HALF), lambda i: (i + nb, 0)),
            pl.BlockSpec((BN, HALF), lambda i: (i, 0)),
            pl.BlockSpec((BN, HALF), lambda i: (i + nb, 0)),
            pl.BlockSpec((3 * D, OUT), lambda i: (0, 0)),
            pl.BlockSpec((1, OUT), lambda i: (0, 0)),
        ],
        out_specs=pl.BlockSpec((BN, OUT), lambda i: (i, 0)),
        out_shape=jax.ShapeDtypeStruct((N, OUT), jnp.float32),
    )(x, y1f, y1f, y2f, y2f, W, b2)


def kernel(x, edge_index, adj_values, W, b):
    row = edge_index[0].astype(jnp.int32)
    col = edge_index[1].astype(jnp.int32)
    adj = adj_values.astype(jnp.float32)

    pad = E_PAD - E
    col2 = jnp.concatenate([col, jnp.zeros((pad,), jnp.int32)]).reshape(-1, GROUP)
    row2 = jnp.concatenate([row, jnp.zeros((pad,), jnp.int32)]).reshape(-1, GROUP)
    adjf = jnp.concatenate([adj, jnp.zeros((pad,), jnp.float32)])

    xs = jnp.concatenate([x[:, :HALF], x[:, HALF:]], axis=0)
    y1f, y2f = _spmm2_sc(xs, col2, row2, adjf)
    return _linear_tc(x, y1f, y2f, W, b.reshape(1, OUT))


# speedup vs baseline: 2.3607x; 2.3607x over previous
import functools

import jax
import jax.numpy as jnp
from jax import lax
from jax.experimental import pallas as pl
from jax.experimental.pallas import tpu as pltpu
from jax.experimental.pallas import tpu_sc as plsc

N = 10000
NPAD = 10240
E = 320000
D = 128
HALF = 64
OUT = 128

NC = 2
NT = 16
LANES = 16

GROUP = 128
GROUPS_PER_CHUNK = 8
CHUNK = GROUP * GROUPS_PER_CHUNK
CHUNKS_PER_TILE = 20
EDGES_PER_TILE = CHUNK * CHUNKS_PER_TILE
E_PAD = EDGES_PER_TILE * NT
ROWS_PER_TILE = NPAD // NT


def _spmm2_sc(xs, col2, row2, adjf):
    mesh = plsc.VectorSubcoreMesh(core_axis_name="c", subcore_axis_name="s")
    y_t = jax.ShapeDtypeStruct((NC * NPAD, HALF), jnp.float32)

    @functools.partial(
        pl.kernel,
        out_type=[y_t, y_t],
        mesh=mesh,
        compiler_params=pltpu.CompilerParams(use_tc_tiling_on_sc=False),
        scratch_types=[
            pltpu.VMEM((GROUPS_PER_CHUNK, GROUP), jnp.int32),
            pltpu.VMEM((GROUPS_PER_CHUNK, GROUP), jnp.int32),
            pltpu.VMEM((CHUNK,), jnp.float32),
            pltpu.VMEM((CHUNK, HALF), jnp.float32),
            pltpu.VMEM_SHARED((NPAD, HALF), jnp.float32),
            pltpu.SemaphoreType.DMA,
        ],
    )
    def k(xs_hbm, col_hbm, row_hbm, adj_hbm, y1_hbm, y2_hbm,
          colv, rowv, adjv, buf, acc, sem):
        core = lax.axis_index("c")
        tid = lax.axis_index("s")
        src_off = core * NPAD

        def zero_acc():
            @pl.loop(0, ROWS_PER_TILE)
            def _(r):
                for c in range(HALF // LANES):
                    buf[r, pl.ds(c * LANES, LANES)] = jnp.zeros(
                        (LANES,), jnp.float32)

            pltpu.sync_copy(buf.at[pl.ds(0, ROWS_PER_TILE)],
                            acc.at[pl.ds(tid * ROWS_PER_TILE, ROWS_PER_TILE)])
            plsc.subcore_barrier()

        zero_acc()

        def hop(src_hbm, out_hbm):
            @pl.loop(0, CHUNKS_PER_TILE)
            def _(s):
                base_g = (tid * CHUNKS_PER_TILE + s) * GROUPS_PER_CHUNK
                pltpu.sync_copy(col_hbm.at[pl.ds(base_g, GROUPS_PER_CHUNK)], colv)
                pltpu.sync_copy(row_hbm.at[pl.ds(base_g, GROUPS_PER_CHUNK)], rowv)
                pltpu.sync_copy(adj_hbm.at[pl.ds(base_g * GROUP, CHUNK)], adjv)

                @pl.loop(0, GROUPS_PER_CHUNK)
                def _(j):
                    for c in range(GROUP // LANES):
                        sl = pl.ds(c * LANES, LANES)
                        colv[j, sl] = colv[j, sl] + src_off

                waits = []
                for j in range(GROUPS_PER_CHUNK):
                    waits.append(pltpu.async_copy(
                        src_hbm.at[colv.at[j]],
                        buf.at[pl.ds(j * GROUP, GROUP)], sem))
                for w in waits:
                    w.wait()

                @pl.loop(0, CHUNK // LANES)
                def _(g):
                    av = adjv[pl.ds(g * LANES, LANES)]
                    for l in range(LANES):
                        e = g * LANES + l
                        a = av[l]
                        for c in range(HALF // LANES):
                            sl = pl.ds(c * LANES, LANES)
                            buf[e, sl] = buf[e, sl] * a

                for j in range(GROUPS_PER_CHUNK):
                    pltpu.sync_copy(buf.at[pl.ds(j * GROUP, GROUP)],
                                    acc.at[rowv.at[j]], add=True)

            plsc.subcore_barrier()
            pltpu.sync_copy(
                acc.at[pl.ds(tid * ROWS_PER_TILE, ROWS_PER_TILE)],
                out_hbm.at[pl.ds(src_off + tid * ROWS_PER_TILE, ROWS_PER_TILE)])
            plsc.subcore_barrier()

        hop(xs_hbm, y1_hbm)
        zero_acc()
        hop(y1_hbm, y2_hbm)

    return k(xs, col2, row2, adjf)


def _linear_tc(x, y1f, y2f, W, b2):
    BN = 640

    def body(x_ref, y1a, y1b, y2a, y2b, w_ref, b_ref, o_ref):
        p = jax.lax.Precision.HIGHEST
        acc = jnp.dot(x_ref[...], w_ref[0:D, :], precision=p,
                      preferred_element_type=jnp.float32)
        acc += jnp.dot(y1a[...], w_ref[D:D + HALF, :], precision=p,
                       preferred_element_type=jnp.float32)
        acc += jnp.dot(y1b[...], w_ref[D + HALF:2 * D, :], precision=p,
                       preferred_element_type=jnp.float32)
        acc += jnp.dot(y2a[...], w_ref[2 * D:2 * D + HALF, :], precision=p,
                       preferred_element_type=jnp.float32)
        acc += jnp.dot(y2b[...], w_ref[2 * D + HALF:3 * D, :], precision=p,
                       preferred_element_type=jnp.float32)
        o_ref[...] = acc + b_ref[...]

    nb = NPAD // BN
    return pl.pallas_call(
        body,
        grid=(nb,),
        in_specs=[
            pl.BlockSpec((BN, D), lambda i: (i, 0)),
            pl.BlockSpec((BN, HALF), lambda i: (i, 0)),
            pl.BlockSpec((BN, HALF), lambda i: (i + nb, 0)),
            pl.BlockSpec((BN, HALF), lambda i: (i, 0)),
            pl.BlockSpec((BN, HALF), lambda i: (i + nb, 0)),
            pl.BlockSpec((3 * D, OUT), lambda i: (0, 0)),
            pl.BlockSpec((1, OUT), lambda i: (0, 0)),
        ],
        out_specs=pl.BlockSpec((BN, OUT), lambda i: (i, 0)),
        out_shape=jax.ShapeDtypeStruct((NPAD, OUT), jnp.float32),
    )(x, y1f, y1f, y2f, y2f, W, b2)


def kernel(x, edge_index, adj_values, W, b):
    row = edge_index[0].astype(jnp.int32)
    col = edge_index[1].astype(jnp.int32)
    adj = adj_values.astype(jnp.float32)

    pad = E_PAD - E
    col2 = jnp.concatenate([col, jnp.zeros((pad,), jnp.int32)]).reshape(-1, GROUP)
    row2 = jnp.concatenate([row, jnp.zeros((pad,), jnp.int32)]).reshape(-1, GROUP)
    adjf = jnp.concatenate([adj, jnp.zeros((pad,), jnp.float32)])

    zrows = jnp.zeros((NPAD - N, HALF), jnp.float32)
    xs = jnp.concatenate([x[:, :HALF], zrows, x[:, HALF:], zrows], axis=0)
    y1f, y2f = _spmm2_sc(xs, col2, row2, adjf)
    xp = jnp.concatenate([x, jnp.zeros((NPAD - N, D), jnp.float32)], axis=0)
    return _linear_tc(xp, y1f, y2f, W, b.reshape(1, OUT))[:N]
